# Initial kernel scaffold; baseline (speedup 1.0000x reference)
import jax
import jax.numpy as jnp
from jax.experimental import pallas as pl
from jax.experimental.pallas import tpu as pltpu
from jax.experimental.pallas import tpu_sc as plsc

K = 16
LEAK = 0.2
ROWS = 256
PTS = 512
GATHER_WIN = 128


def _lrelu(x):
    return jnp.where(x >= 0, x, LEAK * x)


def _topk_body(feat_ref, x_ref, idx_ref):
    b = pl.program_id(0)
    ft = feat_ref[0]
    xt = x_ref[0]
    n = ft.shape[1]
    sq_full = jnp.sum(ft * ft, axis=0, keepdims=True)
    sq_tile = jnp.sum(xt * xt, axis=1, keepdims=True)
    dot = jax.lax.dot_general(
        xt, ft, (((1,), (0,)), ((), ())),
        preferred_element_type=jnp.float32,
        precision=jax.lax.Precision.HIGHEST)
    d = sq_tile + sq_full - 2.0 * dot
    iota = jax.lax.broadcasted_iota(jnp.int32, d.shape, 1)
    inf = jnp.float32(jnp.inf)
    cols = []
    for _ in range(K):
        m = jnp.min(d, axis=1, keepdims=True)
        cand = jnp.where(d <= m, iota, n)
        sel = jnp.min(cand, axis=1, keepdims=True)
        cols.append(sel)
        d = jnp.where(cand == sel, inf, d)
    idx_ref[0] = jnp.concatenate(cols, axis=1) + b * n


def _topk_indices(feat, x):
    B, C, N = feat.shape
    return pl.pallas_call(
        _topk_body,
        grid=(B, N // ROWS),
        in_specs=[
            pl.BlockSpec((1, C, N), lambda b, i: (b, 0, 0)),
            pl.BlockSpec((1, ROWS, C), lambda b, i: (b, i, 0)),
        ],
        out_specs=pl.BlockSpec((1, ROWS, K), lambda b, i: (b, i, 0)),
        out_shape=jax.ShapeDtypeStruct((B, N, K), jnp.int32),
        compiler_params=pltpu.CompilerParams(
            dimension_semantics=("arbitrary", "arbitrary")),
    )(feat, x)


def _sc_gather(x2, flat_idx):
    L = flat_idx.shape[1]
    C = x2.shape[1]
    mesh = plsc.VectorSubcoreMesh(core_axis_name="core",
                                  subcore_axis_name="subcore")

    @pl.kernel(out_type=jax.ShapeDtypeStruct((L, C), x2.dtype), mesh=mesh)
    def kern(x_hbm, i_hbm, o_hbm):
        def body(i_vmem, o_vmem):
            pltpu.sync_copy(x_hbm.at[i_vmem.at[0]], o_vmem)

        pltpu.emit_pipeline(
            body,
            grid=(L // GATHER_WIN,),
            in_specs=[pl.BlockSpec((1, GATHER_WIN), lambda i: (0, i))],
            out_specs=[pl.BlockSpec((GATHER_WIN, C), lambda i: (i, 0))],
            core_axis_name=("core", "subcore"),
            dimension_semantics=(pltpu.PARALLEL,),
        )(i_hbm, o_hbm)

    return kern(x2, flat_idx)


def _mlp_body(g_ref, c_ref, wn_ref, we_ref, w1_ref, w2_ref, o_ref):
    g = g_ref[...]
    cen = c_ref[...]
    wn = wn_ref[...]
    we = we_ref[...]
    w1 = w1_ref[...]
    w2 = w2_ref[...]
    p = cen.shape[0]

    def mm(a, b):
        return jax.lax.dot_general(
            a, b, (((1,), (0,)), ((), ())),
            preferred_element_type=jnp.float32)

    node = _lrelu(mm(g, wn))
    ce = mm(cen, we)
    ce_rep = jnp.broadcast_to(ce[:, None, :], (p, K, ce.shape[1]))
    ce_rep = ce_rep.reshape(p * K, ce.shape[1])
    edge = _lrelu(mm(g, we) - ce_rep)
    h = _lrelu(mm(node + edge, w1))
    h = _lrelu(mm(h, w2))
    o_ref[...] = jnp.max(h.reshape(p, K, h.shape[1]), axis=1)


def _mlp_max(g, x2, wn, we, w1, w2):
    M = x2.shape[0]
    return pl.pallas_call(
        _mlp_body,
        grid=(M // PTS,),
        in_specs=[
            pl.BlockSpec((PTS * K, 32), lambda i: (i, 0)),
            pl.BlockSpec((PTS, 32), lambda i: (i, 0)),
            pl.BlockSpec((32, 32), lambda i: (0, 0)),
            pl.BlockSpec((32, 32), lambda i: (0, 0)),
            pl.BlockSpec((32, 32), lambda i: (0, 0)),
            pl.BlockSpec((32, 64), lambda i: (0, 0)),
        ],
        out_specs=pl.BlockSpec((PTS, 64), lambda i: (i, 0)),
        out_shape=jax.ShapeDtypeStruct((M, 64), jnp.float32),
        compiler_params=pltpu.CompilerParams(
            dimension_semantics=("arbitrary",)),
    )(g, x2, wn, we, w1, w2)


def kernel(feat, W_node, W_edge, W_mlp1, W_mlp2):
    B, C, N = feat.shape
    x = jnp.transpose(feat, (0, 2, 1))
    idx = _topk_indices(feat, x)
    x2 = x.reshape(B * N, C)
    flat_idx = idx.reshape(1, B * N * K)
    g = _sc_gather(x2, flat_idx)
    out = _mlp_max(g, x2, W_node.T, W_edge.T, W_mlp1.T, W_mlp2.T)
    return out.reshape(B, N, 64).transpose(0, 2, 1)[..., None]


# speedup vs baseline: 16.8167x; 16.8167x over previous
import jax
import jax.numpy as jnp
from jax.experimental import pallas as pl
from jax.experimental.pallas import tpu as pltpu
from jax.experimental.pallas import tpu_sc as plsc

K = 16
LEAK = 0.2
ROWS = 256
PTS = 512
GATHER_WIN = 128


def _lrelu(x):
    return jnp.where(x >= 0, x, LEAK * x)


def _topk_body(feat_ref, x_ref, idx_ref):
    b = pl.program_id(0)
    ft = feat_ref[0]
    xt = x_ref[0]
    n = ft.shape[1]
    sq_full = jnp.sum(ft * ft, axis=0, keepdims=True)
    sq_tile = jnp.sum(xt * xt, axis=1, keepdims=True)
    dot = jax.lax.dot_general(
        xt, ft, (((1,), (0,)), ((), ())),
        preferred_element_type=jnp.float32)
    d = sq_tile + sq_full - 2.0 * dot
    iota = jax.lax.broadcasted_iota(jnp.int32, d.shape, 1)
    inf = jnp.float32(jnp.inf)
    cols = []
    for _ in range(K):
        m = jnp.min(d, axis=1, keepdims=True)
        cand = jnp.where(d <= m, iota, n)
        sel = jnp.min(cand, axis=1, keepdims=True)
        cols.append(sel)
        d = jnp.where(cand == sel, inf, d)
    idx_ref[0] = jnp.concatenate(cols, axis=1) + b * n


def _topk_indices(feat, x):
    B, C, N = feat.shape
    return pl.pallas_call(
        _topk_body,
        grid=(B, N // ROWS),
        in_specs=[
            pl.BlockSpec((1, C, N), lambda b, i: (b, 0, 0)),
            pl.BlockSpec((1, ROWS, C), lambda b, i: (b, i, 0)),
        ],
        out_specs=pl.BlockSpec((1, ROWS, K), lambda b, i: (b, i, 0)),
        out_shape=jax.ShapeDtypeStruct((B, N, K), jnp.int32),
        compiler_params=pltpu.CompilerParams(
            dimension_semantics=("arbitrary", "arbitrary")),
    )(feat, x)


def _sc_gather(x2, flat_idx):
    L = flat_idx.shape[1]
    C = x2.shape[1]
    mesh = plsc.VectorSubcoreMesh(core_axis_name="core",
                                  subcore_axis_name="subcore")

    @pl.kernel(out_type=jax.ShapeDtypeStruct((L, C), x2.dtype), mesh=mesh)
    def kern(x_hbm, i_hbm, o_hbm):
        def body(i_vmem, o_vmem):
            pltpu.sync_copy(x_hbm.at[i_vmem.at[0]], o_vmem)

        pltpu.emit_pipeline(
            body,
            grid=(L // GATHER_WIN,),
            in_specs=[pl.BlockSpec((1, GATHER_WIN), lambda i: (0, i))],
            out_specs=[pl.BlockSpec((GATHER_WIN, C), lambda i: (i, 0))],
            core_axis_name=("core", "subcore"),
            dimension_semantics=(pltpu.PARALLEL,),
        )(i_hbm, o_hbm)

    return kern(x2, flat_idx)


def _mlp_body(g_ref, c_ref, wn_ref, we_ref, w1_ref, w2_ref, o_ref):
    g = g_ref[...][:, :32]
    cen = c_ref[...]
    wn = wn_ref[...]
    we = we_ref[...]
    w1 = w1_ref[...]
    w2 = w2_ref[...]
    p = cen.shape[0]

    def mm(a, b):
        return jax.lax.dot_general(
            a, b, (((1,), (0,)), ((), ())),
            preferred_element_type=jnp.float32)

    node = _lrelu(mm(g, wn))
    ce = mm(cen, we)
    ce_rep = jnp.broadcast_to(ce[:, None, :], (p, K, ce.shape[1]))
    ce_rep = ce_rep.reshape(p * K, ce.shape[1])
    edge = _lrelu(mm(g, we) - ce_rep)
    h = _lrelu(mm(node + edge, w1))
    h = _lrelu(mm(h, w2))
    o_ref[...] = jnp.max(h.reshape(p, K, h.shape[1]), axis=1)


def _mlp_max(g, x2, wn, we, w1, w2):
    M = x2.shape[0]
    return pl.pallas_call(
        _mlp_body,
        grid=(M // PTS,),
        in_specs=[
            pl.BlockSpec((PTS * K, 128), lambda i: (i, 0)),
            pl.BlockSpec((PTS, 32), lambda i: (i, 0)),
            pl.BlockSpec((32, 32), lambda i: (0, 0)),
            pl.BlockSpec((32, 32), lambda i: (0, 0)),
            pl.BlockSpec((32, 32), lambda i: (0, 0)),
            pl.BlockSpec((32, 64), lambda i: (0, 0)),
        ],
        out_specs=pl.BlockSpec((PTS, 64), lambda i: (i, 0)),
        out_shape=jax.ShapeDtypeStruct((M, 64), jnp.float32),
        compiler_params=pltpu.CompilerParams(
            dimension_semantics=("arbitrary",)),
    )(g, x2, wn, we, w1, w2)


def kernel(feat, W_node, W_edge, W_mlp1, W_mlp2):
    B, C, N = feat.shape
    x = jnp.transpose(feat, (0, 2, 1))
    idx = _topk_indices(feat, x)
    x2 = x.reshape(B * N, C)
    flat_idx = idx.reshape(1, B * N * K)
    x2p = jnp.pad(x2, ((0, 0), (0, 128 - C)))
    g = _sc_gather(x2p, flat_idx)
    out = _mlp_max(g, x2, W_node.T, W_edge.T, W_mlp1.T, W_mlp2.T)
    return out.reshape(B, N, 64).transpose(0, 2, 1)[..., None]


# speedup vs baseline: 18.0916x; 1.0758x over previous
import jax
import jax.numpy as jnp
from jax.experimental import pallas as pl
from jax.experimental.pallas import tpu as pltpu
from jax.experimental.pallas import tpu_sc as plsc

K = 16
LEAK = 0.2
ROWS = 256
PTS = 512
GATHER_WIN = 128


def _lrelu(x):
    return jnp.where(x >= 0, x, LEAK * x)


def _topk_body(feat_ref, x_ref, idx_ref):
    b = pl.program_id(0)
    ft = feat_ref[0]
    xt = x_ref[0]
    n = ft.shape[1]
    sq_full = jnp.sum(ft * ft, axis=0, keepdims=True)
    sq_tile = jnp.sum(xt * xt, axis=1, keepdims=True)
    dot = jax.lax.dot_general(
        xt, ft, (((1,), (0,)), ((), ())),
        preferred_element_type=jnp.float32)
    d = sq_tile + sq_full - 2.0 * dot
    iota = jax.lax.broadcasted_iota(jnp.int32, d.shape, 1)
    inf = jnp.float32(jnp.inf)
    cols = []
    for _ in range(K):
        sel = jnp.argmin(d, axis=1).astype(jnp.int32)[:, None]
        cols.append(sel)
        d = jnp.where(iota == sel, inf, d)
    idx_ref[0] = jnp.concatenate(cols, axis=1) + b * n


def _topk_indices(feat, x):
    B, C, N = feat.shape
    return pl.pallas_call(
        _topk_body,
        grid=(B, N // ROWS),
        in_specs=[
            pl.BlockSpec((1, C, N), lambda b, i: (b, 0, 0)),
            pl.BlockSpec((1, ROWS, C), lambda b, i: (b, i, 0)),
        ],
        out_specs=pl.BlockSpec((1, ROWS, K), lambda b, i: (b, i, 0)),
        out_shape=jax.ShapeDtypeStruct((B, N, K), jnp.int32),
        compiler_params=pltpu.CompilerParams(
            dimension_semantics=("arbitrary", "arbitrary")),
    )(feat, x)


def _sc_gather(x2, flat_idx):
    L = flat_idx.shape[1]
    C = x2.shape[1]
    mesh = plsc.VectorSubcoreMesh(core_axis_name="core",
                                  subcore_axis_name="subcore")

    @pl.kernel(out_type=jax.ShapeDtypeStruct((L, C), x2.dtype), mesh=mesh)
    def kern(x_hbm, i_hbm, o_hbm):
        def body(i_vmem, o_vmem):
            pltpu.sync_copy(x_hbm.at[i_vmem.at[0]], o_vmem)

        pltpu.emit_pipeline(
            body,
            grid=(L // GATHER_WIN,),
            in_specs=[pl.BlockSpec((1, GATHER_WIN), lambda i: (0, i))],
            out_specs=[pl.BlockSpec((GATHER_WIN, C), lambda i: (i, 0))],
            core_axis_name=("core", "subcore"),
            dimension_semantics=(pltpu.PARALLEL,),
        )(i_hbm, o_hbm)

    return kern(x2, flat_idx)


def _mlp_body(g_ref, c_ref, wn_ref, we_ref, w1_ref, w2_ref, o_ref):
    g = g_ref[...][:, :32]
    cen = c_ref[...]
    wn = wn_ref[...]
    we = we_ref[...]
    w1 = w1_ref[...]
    w2 = w2_ref[...]
    p = cen.shape[0]

    def mm(a, b):
        return jax.lax.dot_general(
            a, b, (((1,), (0,)), ((), ())),
            preferred_element_type=jnp.float32)

    node = _lrelu(mm(g, wn))
    ce = mm(cen, we)
    ce_rep = jnp.broadcast_to(ce[:, None, :], (p, K, ce.shape[1]))
    ce_rep = ce_rep.reshape(p * K, ce.shape[1])
    edge = _lrelu(mm(g, we) - ce_rep)
    h = _lrelu(mm(node + edge, w1))
    h = _lrelu(mm(h, w2))
    o_ref[...] = jnp.max(h.reshape(p, K, h.shape[1]), axis=1)


def _mlp_max(g, x2, wn, we, w1, w2):
    M = x2.shape[0]
    return pl.pallas_call(
        _mlp_body,
        grid=(M // PTS,),
        in_specs=[
            pl.BlockSpec((PTS * K, 128), lambda i: (i, 0)),
            pl.BlockSpec((PTS, 32), lambda i: (i, 0)),
            pl.BlockSpec((32, 32), lambda i: (0, 0)),
            pl.BlockSpec((32, 32), lambda i: (0, 0)),
            pl.BlockSpec((32, 32), lambda i: (0, 0)),
            pl.BlockSpec((32, 64), lambda i: (0, 0)),
        ],
        out_specs=pl.BlockSpec((PTS, 64), lambda i: (i, 0)),
        out_shape=jax.ShapeDtypeStruct((M, 64), jnp.float32),
        compiler_params=pltpu.CompilerParams(
            dimension_semantics=("arbitrary",)),
    )(g, x2, wn, we, w1, w2)


def kernel(feat, W_node, W_edge, W_mlp1, W_mlp2):
    B, C, N = feat.shape
    x = jnp.transpose(feat, (0, 2, 1))
    idx = _topk_indices(feat, x)
    x2 = x.reshape(B * N, C)
    flat_idx = idx.reshape(1, B * N * K)
    x2p = jnp.pad(x2, ((0, 0), (0, 128 - C)))
    g = _sc_gather(x2p, flat_idx)
    out = _mlp_max(g, x2, W_node.T, W_edge.T, W_mlp1.T, W_mlp2.T)
    return out.reshape(B, N, 64).transpose(0, 2, 1)[..., None]


# speedup vs baseline: 18.5358x; 1.0246x over previous
import jax
import jax.numpy as jnp
from jax.experimental import pallas as pl
from jax.experimental.pallas import tpu as pltpu
from jax.experimental.pallas import tpu_sc as plsc

K = 16
LEAK = 0.2
ROWS = 256
PTS = 512
GATHER_WIN = 128


def _lrelu(x):
    return jnp.where(x >= 0, x, LEAK * x)


def _topk_body(feat_ref, x_ref, idx_ref):
    ft = feat_ref[...]
    xt = x_ref[...]
    n = ft.shape[1]
    sq_full = jnp.sum(ft * ft, axis=0, keepdims=True)
    sq_tile = jnp.sum(xt * xt, axis=1, keepdims=True)
    dot = jax.lax.dot_general(
        xt, ft, (((1,), (0,)), ((), ())),
        preferred_element_type=jnp.float32)
    d = sq_tile + sq_full - 2.0 * dot
    iota = jax.lax.broadcasted_iota(jnp.int32, d.shape, 1)
    inf = jnp.float32(jnp.inf)
    cols = []
    for _ in range(K):
        sel = jnp.argmin(d, axis=1).astype(jnp.int32)[:, None]
        cols.append(sel)
        d = jnp.where(iota == sel, inf, d)
    idx_ref[...] = jnp.concatenate(cols, axis=1)


def _topk_indices(feat_b, x_b):
    C, N = feat_b.shape
    return pl.pallas_call(
        _topk_body,
        grid=(N // ROWS,),
        in_specs=[
            pl.BlockSpec((C, N), lambda i: (0, 0)),
            pl.BlockSpec((ROWS, C), lambda i: (i, 0)),
        ],
        out_specs=pl.BlockSpec((ROWS, K), lambda i: (i, 0)),
        out_shape=jax.ShapeDtypeStruct((N, K), jnp.int32),
        compiler_params=pltpu.CompilerParams(
            dimension_semantics=("arbitrary",)),
    )(feat_b, x_b)


def _sc_gather(x2, flat_idx):
    L = flat_idx.shape[1]
    C = x2.shape[1]
    mesh = plsc.VectorSubcoreMesh(core_axis_name="core",
                                  subcore_axis_name="subcore")

    @pl.kernel(out_type=jax.ShapeDtypeStruct((L, C), x2.dtype), mesh=mesh)
    def kern(x_hbm, i_hbm, o_hbm):
        def body(i_vmem, o_vmem):
            pltpu.sync_copy(x_hbm.at[i_vmem.at[0]], o_vmem)

        pltpu.emit_pipeline(
            body,
            grid=(L // GATHER_WIN,),
            in_specs=[pl.BlockSpec((1, GATHER_WIN), lambda i: (0, i))],
            out_specs=[pl.BlockSpec((GATHER_WIN, C), lambda i: (i, 0))],
            core_axis_name=("core", "subcore"),
            dimension_semantics=(pltpu.PARALLEL,),
        )(i_hbm, o_hbm)

    return kern(x2, flat_idx)


def _mlp_body(g_ref, c_ref, wn_ref, we_ref, w1_ref, w2_ref, o_ref):
    g = g_ref[...][:, :32]
    cen = c_ref[...]
    wn = wn_ref[...]
    we = we_ref[...]
    w1 = w1_ref[...]
    w2 = w2_ref[...]
    p = cen.shape[0]

    def mm(a, b):
        return jax.lax.dot_general(
            a, b, (((1,), (0,)), ((), ())),
            preferred_element_type=jnp.float32)

    node = _lrelu(mm(g, wn))
    ce = mm(cen, we)
    ce_rep = jnp.broadcast_to(ce[:, None, :], (p, K, ce.shape[1]))
    ce_rep = ce_rep.reshape(p * K, ce.shape[1])
    edge = _lrelu(mm(g, we) - ce_rep)
    h = _lrelu(mm(node + edge, w1))
    h = _lrelu(mm(h, w2))
    o_ref[...] = jnp.max(h.reshape(p, K, h.shape[1]), axis=1)


def _mlp_max(g, x2, wn, we, w1, w2):
    M = x2.shape[0]
    return pl.pallas_call(
        _mlp_body,
        grid=(M // PTS,),
        in_specs=[
            pl.BlockSpec((PTS * K, 128), lambda i: (i, 0)),
            pl.BlockSpec((PTS, 32), lambda i: (i, 0)),
            pl.BlockSpec((32, 32), lambda i: (0, 0)),
            pl.BlockSpec((32, 32), lambda i: (0, 0)),
            pl.BlockSpec((32, 32), lambda i: (0, 0)),
            pl.BlockSpec((32, 64), lambda i: (0, 0)),
        ],
        out_specs=pl.BlockSpec((PTS, 64), lambda i: (i, 0)),
        out_shape=jax.ShapeDtypeStruct((M, 64), jnp.float32),
        compiler_params=pltpu.CompilerParams(
            dimension_semantics=("arbitrary",)),
    )(g, x2, wn, we, w1, w2)


def kernel(feat, W_node, W_edge, W_mlp1, W_mlp2):
    B, C, N = feat.shape
    x = jnp.transpose(feat, (0, 2, 1))
    x2 = x.reshape(B * N, C)
    x2p = jnp.pad(x2, ((0, 0), (0, 128 - C)))
    idxs = [_topk_indices(feat[b], x[b]) for b in range(B)]
    gs = [_sc_gather(x2p, (idxs[b] + b * N).reshape(1, N * K))
          for b in range(B)]
    outs = [_mlp_max(gs[b], x[b], W_node.T, W_edge.T, W_mlp1.T, W_mlp2.T)
            for b in range(B)]
    out = jnp.stack(outs)
    return out.transpose(0, 2, 1)[..., None]


# speedup vs baseline: 28.7401x; 1.5505x over previous
import jax
import jax.numpy as jnp
from jax.experimental import pallas as pl
from jax.experimental.pallas import tpu as pltpu
from jax.experimental.pallas import tpu_sc as plsc

K = 16
LEAK = 0.2
ROWS = 128
PTS = 512
GATHER_WIN = 128


def _lrelu(x):
    return jnp.where(x >= 0, x, LEAK * x)


def _topk_body(feat_ref, x_ref, idx_ref):
    ft = feat_ref[...]
    xt = x_ref[...]
    n = ft.shape[1]
    sq_full = jnp.sum(ft * ft, axis=0, keepdims=True)
    sq_tile = jnp.sum(xt * xt, axis=1, keepdims=True)
    dot = jax.lax.dot_general(
        xt, ft, (((1,), (0,)), ((), ())),
        preferred_element_type=jnp.float32)
    d = sq_tile + sq_full - 2.0 * dot
    r = d.shape[0]
    inf = jnp.float32(jnp.inf)
    nslices = n // 512
    iota512 = jax.lax.broadcasted_iota(jnp.int32, (r, 512), 1)

    vs, js = [], []
    work = d
    for rnd in range(4):
        fv = work[:, 0:512]
        fi = iota512
        for s in range(1, nslices):
            a = work[:, s * 512:(s + 1) * 512]
            c = a < fv
            fv = jnp.where(c, a, fv)
            fi = jnp.where(c, iota512 + s * 512, fi)
        vs.append(fv)
        js.append(fi)
        if rnd < 3:
            ft = jnp.concatenate([fv] * nslices, axis=1)
            work = jnp.where(work == ft, inf, work)

    v0, v1, v2, v3 = vs
    j0, j1, j2, j3 = js
    cols = []
    m = None
    for _ in range(K):
        m = jnp.min(v0, axis=1, keepdims=True)
        cand = jnp.where(v0 <= m, iota512, 512)
        s = jnp.min(cand, axis=1, keepdims=True)
        pop = cand == s
        cols.append(jnp.min(jnp.where(pop, j0, n), axis=1, keepdims=True))
        v0 = jnp.where(pop, v1, v0)
        j0 = jnp.where(pop, j1, j0)
        v1 = jnp.where(pop, v2, v1)
        j1 = jnp.where(pop, j2, j1)
        v2 = jnp.where(pop, v3, v2)
        j2 = jnp.where(pop, j3, j2)
        v3 = jnp.where(pop, inf, v3)
    idx_fold = jnp.concatenate(cols, axis=1)

    count = jnp.sum((d <= m).astype(jnp.int32), axis=1)
    ok = jnp.all(count == K)

    def _slow():
        dd = d
        iota = jax.lax.broadcasted_iota(jnp.int32, dd.shape, 1)
        out = []
        for _ in range(K):
            sel = jnp.argmin(dd, axis=1).astype(jnp.int32)[:, None]
            out.append(sel)
            dd = jnp.where(iota == sel, inf, dd)
        return jnp.concatenate(out, axis=1)

    idx_ref[...] = jax.lax.cond(ok, lambda: idx_fold, _slow)


def _topk_indices(feat_b, x_b):
    C, N = feat_b.shape
    return pl.pallas_call(
        _topk_body,
        grid=(N // ROWS,),
        in_specs=[
            pl.BlockSpec((C, N), lambda i: (0, 0)),
            pl.BlockSpec((ROWS, C), lambda i: (i, 0)),
        ],
        out_specs=pl.BlockSpec((ROWS, K), lambda i: (i, 0)),
        out_shape=jax.ShapeDtypeStruct((N, K), jnp.int32),
        compiler_params=pltpu.CompilerParams(
            dimension_semantics=("arbitrary",)),
    )(feat_b, x_b)


def _sc_gather(x2, flat_idx):
    L = flat_idx.shape[1]
    C = x2.shape[1]
    mesh = plsc.VectorSubcoreMesh(core_axis_name="core",
                                  subcore_axis_name="subcore")

    @pl.kernel(out_type=jax.ShapeDtypeStruct((L, C), x2.dtype), mesh=mesh)
    def kern(x_hbm, i_hbm, o_hbm):
        def body(i_vmem, o_vmem):
            pltpu.sync_copy(x_hbm.at[i_vmem.at[0]], o_vmem)

        pltpu.emit_pipeline(
            body,
            grid=(L // GATHER_WIN,),
            in_specs=[pl.BlockSpec((1, GATHER_WIN), lambda i: (0, i))],
            out_specs=[pl.BlockSpec((GATHER_WIN, C), lambda i: (i, 0))],
            core_axis_name=("core", "subcore"),
            dimension_semantics=(pltpu.PARALLEL,),
        )(i_hbm, o_hbm)

    return kern(x2, flat_idx)


def _mlp_body(g_ref, c_ref, wn_ref, we_ref, w1_ref, w2_ref, o_ref):
    g = g_ref[...][:, :32]
    cen = c_ref[...]
    wn = wn_ref[...]
    we = we_ref[...]
    w1 = w1_ref[...]
    w2 = w2_ref[...]
    p = cen.shape[0]

    def mm(a, b):
        return jax.lax.dot_general(
            a, b, (((1,), (0,)), ((), ())),
            preferred_element_type=jnp.float32)

    node = _lrelu(mm(g, wn))
    ce = mm(cen, we)
    ce_rep = jnp.broadcast_to(ce[:, None, :], (p, K, ce.shape[1]))
    ce_rep = ce_rep.reshape(p * K, ce.shape[1])
    edge = _lrelu(mm(g, we) - ce_rep)
    h = _lrelu(mm(node + edge, w1))
    h = _lrelu(mm(h, w2))
    o_ref[...] = jnp.max(h.reshape(p, K, h.shape[1]), axis=1)


def _mlp_max(g, x2, wn, we, w1, w2):
    M = x2.shape[0]
    return pl.pallas_call(
        _mlp_body,
        grid=(M // PTS,),
        in_specs=[
            pl.BlockSpec((PTS * K, 128), lambda i: (i, 0)),
            pl.BlockSpec((PTS, 32), lambda i: (i, 0)),
            pl.BlockSpec((32, 32), lambda i: (0, 0)),
            pl.BlockSpec((32, 32), lambda i: (0, 0)),
            pl.BlockSpec((32, 32), lambda i: (0, 0)),
            pl.BlockSpec((32, 64), lambda i: (0, 0)),
        ],
        out_specs=pl.BlockSpec((PTS, 64), lambda i: (i, 0)),
        out_shape=jax.ShapeDtypeStruct((M, 64), jnp.float32),
        compiler_params=pltpu.CompilerParams(
            dimension_semantics=("arbitrary",)),
    )(g, x2, wn, we, w1, w2)


def kernel(feat, W_node, W_edge, W_mlp1, W_mlp2):
    B, C, N = feat.shape
    x = jnp.transpose(feat, (0, 2, 1))
    x2 = x.reshape(B * N, C)
    x2p = jnp.pad(x2, ((0, 0), (0, 128 - C)))
    idxs = [_topk_indices(feat[b], x[b]) for b in range(B)]
    gs = [_sc_gather(x2p, (idxs[b] + b * N).reshape(1, N * K))
          for b in range(B)]
    outs = [_mlp_max(gs[b], x[b], W_node.T, W_edge.T, W_mlp1.T, W_mlp2.T)
            for b in range(B)]
    out = jnp.stack(outs)
    return out.transpose(0, 2, 1)[..., None]


# speedup vs baseline: 29.4382x; 1.0243x over previous
import jax
import jax.numpy as jnp
from jax.experimental import pallas as pl
from jax.experimental.pallas import tpu as pltpu
from jax.experimental.pallas import tpu_sc as plsc

K = 16
LEAK = 0.2
ROWS = 256
PTS = 512
GATHER_WIN = 128


def _lrelu(x):
    return jnp.where(x >= 0, x, LEAK * x)


def _topk_body(feat_ref, x_ref, idx_ref):
    ft = feat_ref[...]
    xt = x_ref[...]
    n = ft.shape[1]
    sq_full = jnp.sum(ft * ft, axis=0, keepdims=True)
    sq_tile = jnp.sum(xt * xt, axis=1, keepdims=True)
    dot = jax.lax.dot_general(
        xt, ft, (((1,), (0,)), ((), ())),
        preferred_element_type=jnp.float32)
    d = sq_tile + sq_full - 2.0 * dot
    r = d.shape[0]
    inf = jnp.float32(jnp.inf)
    nslices = n // 512
    iota512 = jax.lax.broadcasted_iota(jnp.int32, (r, 512), 1)

    infs = jnp.full((r, 512), inf, jnp.float32)
    v0, j0 = d[:, 0:512], iota512
    v1, j1 = infs, iota512
    v2, j2 = infs, iota512
    v3, j3 = infs, iota512
    for s in range(1, nslices):
        t = d[:, s * 512:(s + 1) * 512]
        ti = iota512 + s * 512
        c = t < v0
        v0, t = jnp.where(c, t, v0), jnp.where(c, v0, t)
        j0, ti = jnp.where(c, ti, j0), jnp.where(c, j0, ti)
        c = t < v1
        v1, t = jnp.where(c, t, v1), jnp.where(c, v1, t)
        j1, ti = jnp.where(c, ti, j1), jnp.where(c, j1, ti)
        c = t < v2
        v2, t = jnp.where(c, t, v2), jnp.where(c, v2, t)
        j2, ti = jnp.where(c, ti, j2), jnp.where(c, j2, ti)
        c = t < v3
        v3 = jnp.where(c, t, v3)
        j3 = jnp.where(c, ti, j3)

    cols = []
    m = None
    for _ in range(K):
        m = jnp.min(v0, axis=1, keepdims=True)
        cand = jnp.where(v0 <= m, iota512, 512)
        s = jnp.min(cand, axis=1, keepdims=True)
        pop = cand == s
        cols.append(jnp.min(jnp.where(pop, j0, n), axis=1, keepdims=True))
        v0 = jnp.where(pop, v1, v0)
        j0 = jnp.where(pop, j1, j0)
        v1 = jnp.where(pop, v2, v1)
        j1 = jnp.where(pop, j2, j1)
        v2 = jnp.where(pop, v3, v2)
        j2 = jnp.where(pop, j3, j2)
        v3 = jnp.where(pop, inf, v3)
    idx_fold = jnp.concatenate(cols, axis=1)

    count = jnp.sum((d <= m).astype(jnp.int32), axis=1)
    ok = jnp.all(count == K)

    def _slow():
        dd = d
        iota = jax.lax.broadcasted_iota(jnp.int32, dd.shape, 1)
        out = []
        for _ in range(K):
            sel = jnp.argmin(dd, axis=1).astype(jnp.int32)[:, None]
            out.append(sel)
            dd = jnp.where(iota == sel, inf, dd)
        return jnp.concatenate(out, axis=1)

    idx_ref[...] = jax.lax.cond(ok, lambda: idx_fold, _slow)


def _topk_indices(feat_b, x_b):
    C, N = feat_b.shape
    return pl.pallas_call(
        _topk_body,
        grid=(N // ROWS,),
        in_specs=[
            pl.BlockSpec((C, N), lambda i: (0, 0)),
            pl.BlockSpec((ROWS, C), lambda i: (i, 0)),
        ],
        out_specs=pl.BlockSpec((ROWS, K), lambda i: (i, 0)),
        out_shape=jax.ShapeDtypeStruct((N, K), jnp.int32),
        compiler_params=pltpu.CompilerParams(
            dimension_semantics=("arbitrary",)),
    )(feat_b, x_b)


def _sc_gather(x2, flat_idx):
    L = flat_idx.shape[1]
    C = x2.shape[1]
    mesh = plsc.VectorSubcoreMesh(core_axis_name="core",
                                  subcore_axis_name="subcore")

    @pl.kernel(out_type=jax.ShapeDtypeStruct((L, C), x2.dtype), mesh=mesh)
    def kern(x_hbm, i_hbm, o_hbm):
        def body(i_vmem, o_vmem):
            pltpu.sync_copy(x_hbm.at[i_vmem.at[0]], o_vmem)

        pltpu.emit_pipeline(
            body,
            grid=(L // GATHER_WIN,),
            in_specs=[pl.BlockSpec((1, GATHER_WIN), lambda i: (0, i))],
            out_specs=[pl.BlockSpec((GATHER_WIN, C), lambda i: (i, 0))],
            core_axis_name=("core", "subcore"),
            dimension_semantics=(pltpu.PARALLEL,),
        )(i_hbm, o_hbm)

    return kern(x2, flat_idx)


def _mlp_body(g_ref, c_ref, wn_ref, we_ref, w1_ref, w2_ref, o_ref):
    g = g_ref[...][:, :32]
    cen = c_ref[...]
    wn = wn_ref[...]
    we = we_ref[...]
    w1 = w1_ref[...]
    w2 = w2_ref[...]
    p = cen.shape[0]

    def mm(a, b):
        return jax.lax.dot_general(
            a, b, (((1,), (0,)), ((), ())),
            preferred_element_type=jnp.float32)

    node = _lrelu(mm(g, wn))
    ce = mm(cen, we)
    ce_rep = jnp.broadcast_to(ce[:, None, :], (p, K, ce.shape[1]))
    ce_rep = ce_rep.reshape(p * K, ce.shape[1])
    edge = _lrelu(mm(g, we) - ce_rep)
    h = _lrelu(mm(node + edge, w1))
    h = _lrelu(mm(h, w2))
    o_ref[...] = jnp.max(h.reshape(p, K, h.shape[1]), axis=1)


def _mlp_max(g, x2, wn, we, w1, w2):
    M = x2.shape[0]
    return pl.pallas_call(
        _mlp_body,
        grid=(M // PTS,),
        in_specs=[
            pl.BlockSpec((PTS * K, 128), lambda i: (i, 0)),
            pl.BlockSpec((PTS, 32), lambda i: (i, 0)),
            pl.BlockSpec((32, 32), lambda i: (0, 0)),
            pl.BlockSpec((32, 32), lambda i: (0, 0)),
            pl.BlockSpec((32, 32), lambda i: (0, 0)),
            pl.BlockSpec((32, 64), lambda i: (0, 0)),
        ],
        out_specs=pl.BlockSpec((PTS, 64), lambda i: (i, 0)),
        out_shape=jax.ShapeDtypeStruct((M, 64), jnp.float32),
        compiler_params=pltpu.CompilerParams(
            dimension_semantics=("arbitrary",)),
    )(g, x2, wn, we, w1, w2)


def kernel(feat, W_node, W_edge, W_mlp1, W_mlp2):
    B, C, N = feat.shape
    x = jnp.transpose(feat, (0, 2, 1))
    x2 = x.reshape(B * N, C)
    x2p = jnp.pad(x2, ((0, 0), (0, 128 - C)))
    idxs = [_topk_indices(feat[b], x[b]) for b in range(B)]
    gs = [_sc_gather(x2p, (idxs[b] + b * N).reshape(1, N * K))
          for b in range(B)]
    outs = [_mlp_max(gs[b], x[b], W_node.T, W_edge.T, W_mlp1.T, W_mlp2.T)
            for b in range(B)]
    out = jnp.stack(outs)
    return out.transpose(0, 2, 1)[..., None]


# speedup vs baseline: 31.9630x; 1.0858x over previous
import jax
import jax.numpy as jnp
from jax.experimental import pallas as pl
from jax.experimental.pallas import tpu as pltpu
from jax.experimental.pallas import tpu_sc as plsc

K = 16
LEAK = 0.2
ROWS = 256
PTS = 512
GATHER_WIN = 128


def _lrelu(x):
    return jnp.where(x >= 0, x, LEAK * x)


def _topk_body(feat_ref, x_ref, idx_ref):
    ft = feat_ref[...]
    xt = x_ref[...]
    n = ft.shape[1]
    sq_full = jnp.sum(ft * ft, axis=0, keepdims=True)
    sq_tile = jnp.sum(xt * xt, axis=1, keepdims=True)
    dot = jax.lax.dot_general(
        xt, ft, (((1,), (0,)), ((), ())),
        preferred_element_type=jnp.float32)
    d = sq_tile + sq_full - 2.0 * dot
    r = d.shape[0]
    inf = jnp.float32(jnp.inf)
    nslices = n // 512
    iota512 = jax.lax.broadcasted_iota(jnp.int32, (r, 512), 1)

    infs = jnp.full((r, 512), inf, jnp.float32)
    v0, j0 = d[:, 0:512], iota512
    v1, j1 = infs, iota512
    v2, j2 = infs, iota512
    v3, j3 = infs, iota512
    for s in range(1, nslices):
        t = d[:, s * 512:(s + 1) * 512]
        ti = iota512 + s * 512
        c = t < v0
        v0, t = jnp.where(c, t, v0), jnp.where(c, v0, t)
        j0, ti = jnp.where(c, ti, j0), jnp.where(c, j0, ti)
        c = t < v1
        v1, t = jnp.where(c, t, v1), jnp.where(c, v1, t)
        j1, ti = jnp.where(c, ti, j1), jnp.where(c, j1, ti)
        c = t < v2
        v2, t = jnp.where(c, t, v2), jnp.where(c, v2, t)
        j2, ti = jnp.where(c, ti, j2), jnp.where(c, j2, ti)
        c = t < v3
        v3 = jnp.where(c, t, v3)
        j3 = jnp.where(c, ti, j3)

    hw = 256
    iota256 = jax.lax.broadcasted_iota(jnp.int32, (r, hw), 1)
    lv = [a[:, :hw] for a in (v0, v1, v2, v3)]
    lj = [a[:, :hw] for a in (j0, j1, j2, j3)]
    rv = [a[:, hw:] for a in (v0, v1, v2, v3)]
    rj = [a[:, hw:] for a in (j0, j1, j2, j3)]
    for t, ti in zip(rv, rj):
        for k in range(4):
            c = t < lv[k]
            lv[k], t = jnp.where(c, t, lv[k]), jnp.where(c, lv[k], t)
            lj[k], ti = jnp.where(c, ti, lj[k]), jnp.where(c, lj[k], ti)
    v0, v1, v2, v3 = lv
    j0, j1, j2, j3 = lj

    cols = []
    m = None
    for _ in range(K):
        m = jnp.min(v0, axis=1, keepdims=True)
        cand = jnp.where(v0 <= m, iota256, hw)
        s = jnp.min(cand, axis=1, keepdims=True)
        pop = cand == s
        cols.append(jnp.min(jnp.where(pop, j0, n), axis=1, keepdims=True))
        v0 = jnp.where(pop, v1, v0)
        j0 = jnp.where(pop, j1, j0)
        v1 = jnp.where(pop, v2, v1)
        j1 = jnp.where(pop, j2, j1)
        v2 = jnp.where(pop, v3, v2)
        j2 = jnp.where(pop, j3, j2)
        v3 = jnp.where(pop, inf, v3)
    idx_fold = jnp.concatenate(cols, axis=1)

    count = jnp.sum((d <= m).astype(jnp.int32), axis=1)
    ok = jnp.all(count == K)

    def _slow():
        dd = d
        iota = jax.lax.broadcasted_iota(jnp.int32, dd.shape, 1)
        out = []
        for _ in range(K):
            sel = jnp.argmin(dd, axis=1).astype(jnp.int32)[:, None]
            out.append(sel)
            dd = jnp.where(iota == sel, inf, dd)
        return jnp.concatenate(out, axis=1)

    idx_ref[...] = jax.lax.cond(ok, lambda: idx_fold, _slow)


def _topk_indices(feat_b, x_b):
    C, N = feat_b.shape
    return pl.pallas_call(
        _topk_body,
        grid=(N // ROWS,),
        in_specs=[
            pl.BlockSpec((C, N), lambda i: (0, 0)),
            pl.BlockSpec((ROWS, C), lambda i: (i, 0)),
        ],
        out_specs=pl.BlockSpec((ROWS, K), lambda i: (i, 0)),
        out_shape=jax.ShapeDtypeStruct((N, K), jnp.int32),
        compiler_params=pltpu.CompilerParams(
            dimension_semantics=("arbitrary",)),
    )(feat_b, x_b)


def _sc_gather(x2, flat_idx):
    L = flat_idx.shape[1]
    C = x2.shape[1]
    mesh = plsc.VectorSubcoreMesh(core_axis_name="core",
                                  subcore_axis_name="subcore")

    @pl.kernel(out_type=jax.ShapeDtypeStruct((L, C), x2.dtype), mesh=mesh)
    def kern(x_hbm, i_hbm, o_hbm):
        def body(i_vmem, o_vmem):
            pltpu.sync_copy(x_hbm.at[i_vmem.at[0]], o_vmem)

        pltpu.emit_pipeline(
            body,
            grid=(L // GATHER_WIN,),
            in_specs=[pl.BlockSpec((1, GATHER_WIN), lambda i: (0, i))],
            out_specs=[pl.BlockSpec((GATHER_WIN, C), lambda i: (i, 0))],
            core_axis_name=("core", "subcore"),
            dimension_semantics=(pltpu.PARALLEL,),
        )(i_hbm, o_hbm)

    return kern(x2, flat_idx)


def _mlp_body(g_ref, c_ref, wn_ref, we_ref, w1_ref, w2_ref, o_ref):
    g = g_ref[...][:, :32]
    cen = c_ref[...]
    wn = wn_ref[...]
    we = we_ref[...]
    w1 = w1_ref[...]
    w2 = w2_ref[...]
    p = cen.shape[0]

    def mm(a, b):
        return jax.lax.dot_general(
            a, b, (((1,), (0,)), ((), ())),
            preferred_element_type=jnp.float32)

    node = _lrelu(mm(g, wn))
    ce = mm(cen, we)
    ce_rep = jnp.broadcast_to(ce[:, None, :], (p, K, ce.shape[1]))
    ce_rep = ce_rep.reshape(p * K, ce.shape[1])
    edge = _lrelu(mm(g, we) - ce_rep)
    h = _lrelu(mm(node + edge, w1))
    h = _lrelu(mm(h, w2))
    o_ref[...] = jnp.max(h.reshape(p, K, h.shape[1]), axis=1)


def _mlp_max(g, x2, wn, we, w1, w2):
    M = x2.shape[0]
    return pl.pallas_call(
        _mlp_body,
        grid=(M // PTS,),
        in_specs=[
            pl.BlockSpec((PTS * K, 128), lambda i: (i, 0)),
            pl.BlockSpec((PTS, 32), lambda i: (i, 0)),
            pl.BlockSpec((32, 32), lambda i: (0, 0)),
            pl.BlockSpec((32, 32), lambda i: (0, 0)),
            pl.BlockSpec((32, 32), lambda i: (0, 0)),
            pl.BlockSpec((32, 64), lambda i: (0, 0)),
        ],
        out_specs=pl.BlockSpec((PTS, 64), lambda i: (i, 0)),
        out_shape=jax.ShapeDtypeStruct((M, 64), jnp.float32),
        compiler_params=pltpu.CompilerParams(
            dimension_semantics=("arbitrary",)),
    )(g, x2, wn, we, w1, w2)


def kernel(feat, W_node, W_edge, W_mlp1, W_mlp2):
    B, C, N = feat.shape
    x = jnp.transpose(feat, (0, 2, 1))
    x2 = x.reshape(B * N, C)
    x2p = jnp.pad(x2, ((0, 0), (0, 128 - C)))
    idxs = [_topk_indices(feat[b], x[b]) for b in range(B)]
    gs = [_sc_gather(x2p, (idxs[b] + b * N).reshape(1, N * K))
          for b in range(B)]
    outs = [_mlp_max(gs[b], x[b], W_node.T, W_edge.T, W_mlp1.T, W_mlp2.T)
            for b in range(B)]
    out = jnp.stack(outs)
    return out.transpose(0, 2, 1)[..., None]


# speedup vs baseline: 33.3948x; 1.0448x over previous
import jax
import jax.numpy as jnp
from jax.experimental import pallas as pl
from jax.experimental.pallas import tpu as pltpu
from jax.experimental.pallas import tpu_sc as plsc

K = 16
LEAK = 0.2
ROWS = 256
PTS = 512
GATHER_WIN = 128


def _lrelu(x):
    return jnp.where(x >= 0, x, LEAK * x)


def _topk_body(feat_ref, x_ref, idx_ref):
    ft = feat_ref[...]
    xt = x_ref[...]
    n = ft.shape[1]
    sq_full = jnp.sum(ft * ft, axis=0, keepdims=True)
    sq_tile = jnp.sum(xt * xt, axis=1, keepdims=True)
    dot = jax.lax.dot_general(
        xt, ft, (((1,), (0,)), ((), ())),
        preferred_element_type=jnp.float32)
    d = sq_tile + sq_full - 2.0 * dot
    r = d.shape[0]
    inf = jnp.float32(jnp.inf)
    nslices = n // 512
    iota512 = jax.lax.broadcasted_iota(jnp.int32, (r, 512), 1)

    infs = jnp.full((r, 512), inf, jnp.float32)
    v0, j0 = d[:, 0:512], iota512
    v1, j1 = infs, iota512
    v2, j2 = infs, iota512
    v3, j3 = infs, iota512
    for s in range(1, nslices):
        t = d[:, s * 512:(s + 1) * 512]
        ti = iota512 + s * 512
        c = t < v0
        v0, t = jnp.where(c, t, v0), jnp.where(c, v0, t)
        j0, ti = jnp.where(c, ti, j0), jnp.where(c, j0, ti)
        c = t < v1
        v1, t = jnp.where(c, t, v1), jnp.where(c, v1, t)
        j1, ti = jnp.where(c, ti, j1), jnp.where(c, j1, ti)
        c = t < v2
        v2, t = jnp.where(c, t, v2), jnp.where(c, v2, t)
        j2, ti = jnp.where(c, ti, j2), jnp.where(c, j2, ti)
        c = t < v3
        v3 = jnp.where(c, t, v3)
        j3 = jnp.where(c, ti, j3)

    lv = [v0, v1, v2, v3]
    lj = [j0, j1, j2, j3]
    hw = 512
    for _ in range(2):
        hw //= 2
        rv = [a[:, hw:] for a in lv]
        rj = [a[:, hw:] for a in lj]
        lv = [a[:, :hw] for a in lv]
        lj = [a[:, :hw] for a in lj]
        for t, ti in zip(rv, rj):
            for k in range(4):
                c = t < lv[k]
                lv[k], t = jnp.where(c, t, lv[k]), jnp.where(c, lv[k], t)
                lj[k], ti = jnp.where(c, ti, lj[k]), jnp.where(c, lj[k], ti)
    v0, v1, v2, v3 = lv
    j0, j1, j2, j3 = lj
    iota256 = jax.lax.broadcasted_iota(jnp.int32, (r, hw), 1)

    cols = []
    m = None
    for _ in range(K):
        m = jnp.min(v0, axis=1, keepdims=True)
        cand = jnp.where(v0 <= m, iota256, hw)
        s = jnp.min(cand, axis=1, keepdims=True)
        pop = cand == s
        cols.append(jnp.min(jnp.where(pop, j0, n), axis=1, keepdims=True))
        v0 = jnp.where(pop, v1, v0)
        j0 = jnp.where(pop, j1, j0)
        v1 = jnp.where(pop, v2, v1)
        j1 = jnp.where(pop, j2, j1)
        v2 = jnp.where(pop, v3, v2)
        j2 = jnp.where(pop, j3, j2)
        v3 = jnp.where(pop, inf, v3)
    idx_fold = jnp.concatenate(cols, axis=1)

    count = jnp.sum((d <= m).astype(jnp.int32), axis=1)
    ok = jnp.all(count == K)

    def _slow():
        dd = d
        iota = jax.lax.broadcasted_iota(jnp.int32, dd.shape, 1)
        out = []
        for _ in range(K):
            sel = jnp.argmin(dd, axis=1).astype(jnp.int32)[:, None]
            out.append(sel)
            dd = jnp.where(iota == sel, inf, dd)
        return jnp.concatenate(out, axis=1)

    idx_ref[...] = jax.lax.cond(ok, lambda: idx_fold, _slow)


def _topk_indices(feat_b, x_b):
    C, N = feat_b.shape
    return pl.pallas_call(
        _topk_body,
        grid=(N // ROWS,),
        in_specs=[
            pl.BlockSpec((C, N), lambda i: (0, 0)),
            pl.BlockSpec((ROWS, C), lambda i: (i, 0)),
        ],
        out_specs=pl.BlockSpec((ROWS, K), lambda i: (i, 0)),
        out_shape=jax.ShapeDtypeStruct((N, K), jnp.int32),
        compiler_params=pltpu.CompilerParams(
            dimension_semantics=("arbitrary",)),
    )(feat_b, x_b)


def _sc_gather(x2, flat_idx):
    L = flat_idx.shape[1]
    C = x2.shape[1]
    mesh = plsc.VectorSubcoreMesh(core_axis_name="core",
                                  subcore_axis_name="subcore")

    @pl.kernel(out_type=jax.ShapeDtypeStruct((L, C), x2.dtype), mesh=mesh)
    def kern(x_hbm, i_hbm, o_hbm):
        def body(i_vmem, o_vmem):
            pltpu.sync_copy(x_hbm.at[i_vmem.at[0]], o_vmem)

        pltpu.emit_pipeline(
            body,
            grid=(L // GATHER_WIN,),
            in_specs=[pl.BlockSpec((1, GATHER_WIN), lambda i: (0, i))],
            out_specs=[pl.BlockSpec((GATHER_WIN, C), lambda i: (i, 0))],
            core_axis_name=("core", "subcore"),
            dimension_semantics=(pltpu.PARALLEL,),
        )(i_hbm, o_hbm)

    return kern(x2, flat_idx)


def _mlp_body(g_ref, c_ref, wn_ref, we_ref, w1_ref, w2_ref, o_ref):
    g = g_ref[...][:, :32]
    cen = c_ref[...]
    wn = wn_ref[...]
    we = we_ref[...]
    w1 = w1_ref[...]
    w2 = w2_ref[...]
    p = cen.shape[0]

    def mm(a, b):
        return jax.lax.dot_general(
            a, b, (((1,), (0,)), ((), ())),
            preferred_element_type=jnp.float32)

    node = _lrelu(mm(g, wn))
    ce = mm(cen, we)
    ce_rep = jnp.broadcast_to(ce[:, None, :], (p, K, ce.shape[1]))
    ce_rep = ce_rep.reshape(p * K, ce.shape[1])
    edge = _lrelu(mm(g, we) - ce_rep)
    h = _lrelu(mm(node + edge, w1))
    h = _lrelu(mm(h, w2))
    o_ref[...] = jnp.max(h.reshape(p, K, h.shape[1]), axis=1)


def _mlp_max(g, x2, wn, we, w1, w2):
    M = x2.shape[0]
    return pl.pallas_call(
        _mlp_body,
        grid=(M // PTS,),
        in_specs=[
            pl.BlockSpec((PTS * K, 128), lambda i: (i, 0)),
            pl.BlockSpec((PTS, 32), lambda i: (i, 0)),
            pl.BlockSpec((32, 32), lambda i: (0, 0)),
            pl.BlockSpec((32, 32), lambda i: (0, 0)),
            pl.BlockSpec((32, 32), lambda i: (0, 0)),
            pl.BlockSpec((32, 64), lambda i: (0, 0)),
        ],
        out_specs=pl.BlockSpec((PTS, 64), lambda i: (i, 0)),
        out_shape=jax.ShapeDtypeStruct((M, 64), jnp.float32),
        compiler_params=pltpu.CompilerParams(
            dimension_semantics=("arbitrary",)),
    )(g, x2, wn, we, w1, w2)


def kernel(feat, W_node, W_edge, W_mlp1, W_mlp2):
    B, C, N = feat.shape
    x = jnp.transpose(feat, (0, 2, 1))
    x2 = x.reshape(B * N, C)
    x2p = jnp.pad(x2, ((0, 0), (0, 128 - C)))
    idxs = [_topk_indices(feat[b], x[b]) for b in range(B)]
    gs = [_sc_gather(x2p, (idxs[b] + b * N).reshape(1, N * K))
          for b in range(B)]
    outs = [_mlp_max(gs[b], x[b], W_node.T, W_edge.T, W_mlp1.T, W_mlp2.T)
            for b in range(B)]
    out = jnp.stack(outs)
    return out.transpose(0, 2, 1)[..., None]
